# baseline (speedup 1.0000x reference)
import functools

import jax
import jax.numpy as jnp
import numpy as np
from jax import lax
from jax.experimental import pallas as pl
from jax.experimental.pallas import tpu as pltpu
from jax.experimental.pallas import tpu_sc as plsc

N_NODES = 10000
N_EDGES = 320000
HIDDEN = 128
GRAPHS = 32
LAYERS = 2

NC = 2
NS = 16
NW = NC * NS

CH = 128

EMB_CHUNKS = 3
EMB_PER_W = EMB_CHUNKS * CH
N_PAD = NW * EMB_PER_W

EDGE_CHUNKS = 80
EDGE_PER_W = EDGE_CHUNKS * CH
E_PAD = NW * EDGE_PER_W
NBUF = 2

ACC_ROWS = 10112
ZROWS = ACC_ROWS // NS
CPROWS = 624
CPREM = N_NODES - NS * CPROWS

_sc_mesh = plsc.VectorSubcoreMesh(core_axis_name="c", subcore_axis_name="s")


NI = 4


@functools.partial(
    pl.kernel,
    out_type=[jax.ShapeDtypeStruct((NC, N_NODES, HIDDEN), jnp.float32),
              jax.ShapeDtypeStruct((N_PAD, HIDDEN), jnp.float32)],
    mesh=_sc_mesh,
    scratch_types=(
        [pltpu.VMEM_SHARED((ACC_ROWS, HIDDEN), jnp.float32)]
        + [pltpu.VMEM((2, CH), jnp.int32)] * NI
        + [pltpu.VMEM((CH,), jnp.int32)] * NI
        + [pltpu.VMEM((CH, HIDDEN), jnp.float32)] * NBUF
        + [pltpu.SemaphoreType.DMA] * NI
        + [pltpu.SemaphoreType.DMA] * NI
        + [pltpu.SemaphoreType.DMA] * NBUF
        + [pltpu.SemaphoreType.DMA, pltpu.SemaphoreType.DMA]
    ),
)
def _edge_scatter_emb(emb_hbm, x_hbm, ei_hbm, zeros_hbm, out_hbm, h_hbm,
                      accum, *bufs):
    idxb = bufs[:NI]
    xs = bufs[NI:2 * NI]
    rows = bufs[2 * NI:2 * NI + NBUF]
    isem = bufs[2 * NI + NBUF:3 * NI + NBUF]
    xsem = bufs[3 * NI + NBUF:4 * NI + NBUF]
    gsem = bufs[4 * NI + NBUF:4 * NI + 2 * NBUF]
    ssem = bufs[4 * NI + 2 * NBUF]
    zsem = bufs[4 * NI + 2 * NBUF + 1]
    c = lax.axis_index("c")
    s = lax.axis_index("s")
    wid = s * NC + c

    zslice = accum.at[pl.ds(pl.multiple_of(s * ZROWS, 8), ZROWS)]
    zdesc = pltpu.async_copy(zeros_hbm, zslice, zsem)

    for t in range(NI):
        pltpu.async_copy(ei_hbm.at[wid, t], idxb[t], isem[t])
    for t in range(NI - 1):
        pltpu.make_async_copy(ei_hbm.at[wid, t], idxb[t], isem[t]).wait()
        pltpu.async_copy(x_hbm.at[idxb[t].at[0]], xs[t], xsem[t])
    for b in range(NBUF):
        pltpu.make_async_copy(x_hbm.at[idxb[b].at[0]], xs[b],
                              xsem[b]).wait()
        pltpu.async_copy(emb_hbm.at[xs[b]], rows[b], gsem[b])

    zdesc.wait()
    plsc.subcore_barrier()

    def step(o, carry):
        for t in range(NI):
            j = o * NI + t
            b = t % NBUF
            pltpu.make_async_copy(emb_hbm.at[xs[t]], rows[b],
                                  gsem[b]).wait()
            pltpu.sync_copy(rows[b], accum.at[idxb[t].at[1]], add=True)

            @pl.when(j + NI < EDGE_CHUNKS)
            def _reidx():
                pltpu.async_copy(ei_hbm.at[wid, j + NI], idxb[t], isem[t])

            @pl.when(j + NI - 1 < EDGE_CHUNKS)
            def _rexs():
                tn = (t + NI - 1) % NI
                pltpu.make_async_copy(ei_hbm.at[wid, j + NI - 1],
                                      idxb[tn], isem[tn]).wait()
                pltpu.async_copy(x_hbm.at[idxb[tn].at[0]], xs[tn],
                                 xsem[tn])

            @pl.when(j + NBUF < EDGE_CHUNKS)
            def _regather():
                tg = (t + NBUF) % NI
                pltpu.make_async_copy(x_hbm.at[idxb[tg].at[0]], xs[tg],
                                      xsem[tg]).wait()
                pltpu.async_copy(emb_hbm.at[xs[tg]], rows[b], gsem[b])
        return carry

    lax.fori_loop(0, EDGE_CHUNKS // NI, step, 0)

    hbase = pl.multiple_of(wid * EMB_PER_W, CH)
    for k in range(EMB_CHUNKS):
        pltpu.async_copy(x_hbm.at[pl.ds(hbase + k * CH, CH)], xs[k],
                         isem[k])
    for k in range(NBUF):
        pltpu.make_async_copy(x_hbm.at[pl.ds(hbase + k * CH, CH)], xs[k],
                              isem[k]).wait()
        pltpu.async_copy(emb_hbm.at[xs[k]], rows[k], gsem[k])

    plsc.subcore_barrier()

    r0 = pl.multiple_of(s * CPROWS, 8)
    pltpu.sync_copy(accum.at[pl.ds(r0, CPROWS)],
                    out_hbm.at[c, pl.ds(r0, CPROWS)])

    @pl.when(s == 0)
    def _rem0():
        pltpu.sync_copy(accum.at[pl.ds(NS * CPROWS, CPREM)],
                        out_hbm.at[c, pl.ds(NS * CPROWS, CPREM)])

    for k in range(EMB_CHUNKS):
        b = k % NBUF
        o = pl.multiple_of(hbase + k * CH, CH)
        pltpu.make_async_copy(emb_hbm.at[xs[k]], rows[b], gsem[b]).wait()
        pltpu.sync_copy(rows[b], h_hbm.at[pl.ds(o, CH)])
        kn = k + NBUF
        if kn < EMB_CHUNKS:
            pltpu.make_async_copy(
                x_hbm.at[pl.ds(hbase + kn * CH, CH)], xs[kn],
                isem[kn]).wait()
            pltpu.async_copy(emb_hbm.at[xs[kn]], rows[b], gsem[b])


@functools.partial(
    pl.kernel,
    out_type=jax.ShapeDtypeStruct((NC, N_NODES, HIDDEN), jnp.float32),
    mesh=_sc_mesh,
    scratch_types=(
        [pltpu.VMEM_SHARED((ACC_ROWS, HIDDEN), jnp.float32)]
        + [pltpu.VMEM((2, CH), jnp.int32)] * NI
        + [pltpu.VMEM((CH, HIDDEN), jnp.float32)] * NBUF
        + [pltpu.SemaphoreType.DMA] * NI
        + [pltpu.SemaphoreType.DMA] * NBUF
        + [pltpu.SemaphoreType.DMA, pltpu.SemaphoreType.DMA]
    ),
)
def _edge_scatter(h_hbm, ei_hbm, zeros_hbm, out_hbm, accum, *bufs):
    idxb = bufs[:NI]
    rows = bufs[NI:NI + NBUF]
    isem = bufs[NI + NBUF:2 * NI + NBUF]
    gsem = bufs[2 * NI + NBUF:2 * NI + 2 * NBUF]
    ssem = bufs[2 * NI + 2 * NBUF]
    zsem = bufs[2 * NI + 2 * NBUF + 1]
    c = lax.axis_index("c")
    s = lax.axis_index("s")
    wid = s * NC + c

    zslice = accum.at[pl.ds(pl.multiple_of(s * ZROWS, 8), ZROWS)]
    zdesc = pltpu.async_copy(zeros_hbm, zslice, zsem)

    for t in range(NI):
        pltpu.async_copy(ei_hbm.at[wid, t], idxb[t], isem[t])
    for b in range(NBUF):
        pltpu.make_async_copy(ei_hbm.at[wid, b], idxb[b], isem[b]).wait()
        pltpu.async_copy(h_hbm.at[idxb[b].at[0]], rows[b], gsem[b])

    zdesc.wait()
    plsc.subcore_barrier()

    def step(o, carry):
        for t in range(NI):
            j = o * NI + t
            b = t % NBUF
            pltpu.make_async_copy(h_hbm.at[idxb[t].at[0]], rows[b],
                                  gsem[b]).wait()
            pltpu.sync_copy(rows[b], accum.at[idxb[t].at[1]], add=True)

            @pl.when(j + NI < EDGE_CHUNKS)
            def _reidx():
                pltpu.async_copy(ei_hbm.at[wid, j + NI], idxb[t], isem[t])

            @pl.when(j + NBUF < EDGE_CHUNKS)
            def _regather():
                tn = (t + NBUF) % NI
                pltpu.make_async_copy(ei_hbm.at[wid, j + NBUF], idxb[tn],
                                      isem[tn]).wait()
                pltpu.async_copy(h_hbm.at[idxb[tn].at[0]], rows[b],
                                 gsem[b])
        return carry

    lax.fori_loop(0, EDGE_CHUNKS // NI, step, 0)
    plsc.subcore_barrier()

    r0 = pl.multiple_of(s * CPROWS, 8)
    pltpu.sync_copy(accum.at[pl.ds(r0, CPROWS)],
                    out_hbm.at[c, pl.ds(r0, CPROWS)])

    @pl.when(s == 0)
    def _rem():
        pltpu.sync_copy(accum.at[pl.ds(NS * CPROWS, CPREM)],
                        out_hbm.at[c, pl.ds(NS * CPROWS, CPREM)])


_GRID_R = 1000


def _gru_block(p0, p1, h, wih, whh, bih, bhh):
    xn = p0 + p1
    gi = jnp.dot(xn, wih, preferred_element_type=jnp.float32) + bih
    gh = jnp.dot(h, whh, preferred_element_type=jnp.float32) + bhh
    r = jax.nn.sigmoid(gi[:, :HIDDEN] + gh[:, :HIDDEN])
    z = jax.nn.sigmoid(gi[:, HIDDEN:2 * HIDDEN] + gh[:, HIDDEN:2 * HIDDEN])
    n = jnp.tanh(gi[:, 2 * HIDDEN:] + r * gh[:, 2 * HIDDEN:])
    return (1.0 - z) * n + z * h


def _gru_body(p0_ref, p1_ref, h_ref, wih_ref, whh_ref, bih_ref, bhh_ref,
              out_ref):
    out_ref[...] = _gru_block(p0_ref[0], p1_ref[0], h_ref[...],
                              wih_ref[...], whh_ref[...],
                              bih_ref[...], bhh_ref[...])


_P_SPEC0 = pl.BlockSpec((1, 1000, HIDDEN), lambda i: (0, i, 0))
_P_SPEC1 = pl.BlockSpec((1, 1000, HIDDEN), lambda i: (1, i, 0))


def _gru_tc(part, h, wih_t, whh_t, bih, bhh):
    grid = (N_NODES // _GRID_R,)
    blk = lambda i: (i, 0)
    whole = lambda i: (0, 0)
    return pl.pallas_call(
        _gru_body,
        grid=grid,
        in_specs=[
            _P_SPEC0,
            _P_SPEC1,
            pl.BlockSpec((_GRID_R, HIDDEN), blk),
            pl.BlockSpec((HIDDEN, 3 * HIDDEN), whole),
            pl.BlockSpec((HIDDEN, 3 * HIDDEN), whole),
            pl.BlockSpec((1, 3 * HIDDEN), whole),
            pl.BlockSpec((1, 3 * HIDDEN), whole),
        ],
        out_specs=pl.BlockSpec((_GRID_R, HIDDEN), blk),
        out_shape=jax.ShapeDtypeStruct((N_NODES, HIDDEN), jnp.float32),
    )(part, part, h, wih_t, whh_t, bih, bhh)


def _gru_tail_body(p0_ref, p1_ref, h_ref, wih_ref, whh_ref, bih_ref,
                   bhh_ref, bat_ref, dw_ref, db_ref, cw_ref, cb_ref,
                   out_ref, pooled_ref):
    i = pl.program_id(0)

    @pl.when(i == 0)
    def _init():
        pooled_ref[...] = jnp.full((GRAPHS, HIDDEN), -jnp.inf,
                                   dtype=jnp.float32)

    hn = _gru_block(p0_ref[0], p1_ref[0], h_ref[...],
                    wih_ref[...], whh_ref[...],
                    bih_ref[...], bhh_ref[...])
    hd = jnp.dot(hn, dw_ref[...], preferred_element_type=jnp.float32)
    hd = hd + db_ref[...]
    bat = bat_ref[...]
    neg = jnp.float32(-jnp.inf)
    zero = jnp.float32(0.0)
    for g in range(GRAPHS):
        madd = jnp.where(bat == g, zero, neg)
        m = (hd + madd).max(axis=0, keepdims=True)
        pooled_ref[g:g + 1, :] = jnp.maximum(pooled_ref[g:g + 1, :], m)

    @pl.when(i == pl.num_programs(0) - 1)
    def _fin():
        logits = jnp.dot(pooled_ref[...], cw_ref[...],
                         preferred_element_type=jnp.float32) + cb_ref[...]
        out_ref[...] = jax.nn.sigmoid(logits)


def _gru_tail_tc(part, h, wih_t, whh_t, bih, bhh, bat2d, dw_t, db, cw_t, cb):
    grid = (N_NODES // _GRID_R,)
    blk = lambda i: (i, 0)
    whole = lambda i: (0, 0)
    return pl.pallas_call(
        _gru_tail_body,
        grid=grid,
        in_specs=[
            _P_SPEC0,
            _P_SPEC1,
            pl.BlockSpec((_GRID_R, HIDDEN), blk),
            pl.BlockSpec((HIDDEN, 3 * HIDDEN), whole),
            pl.BlockSpec((HIDDEN, 3 * HIDDEN), whole),
            pl.BlockSpec((1, 3 * HIDDEN), whole),
            pl.BlockSpec((1, 3 * HIDDEN), whole),
            pl.BlockSpec((_GRID_R, 1), blk),
            pl.BlockSpec((HIDDEN, HIDDEN), whole),
            pl.BlockSpec((1, HIDDEN), whole),
            pl.BlockSpec((HIDDEN, 1), whole),
            pl.BlockSpec((1, 1), whole),
        ],
        out_specs=pl.BlockSpec((GRAPHS, 1), whole),
        out_shape=jax.ShapeDtypeStruct((GRAPHS, 1), jnp.float32),
        scratch_shapes=[pltpu.VMEM((GRAPHS, HIDDEN), jnp.float32)],
    )(part, part, h, wih_t, whh_t, bih, bhh, bat2d, dw_t, db, cw_t, cb)


def kernel(x, edge_index, batch, emb, W_ih, W_hh, b_ih, b_hh,
           dense_W, dense_b, clf_W, clf_b):
    x_pad = jnp.concatenate(
        [x, jnp.zeros((N_PAD - N_NODES,), jnp.int32)])

    npad = E_PAD - N_EDGES
    pad_iota = np.arange(npad, dtype=np.int32)
    pad_src = jnp.asarray(pad_iota % N_NODES)
    pad_dst = jnp.asarray(N_NODES + pad_iota % (ACC_ROWS - N_NODES))
    src = jnp.concatenate(
        [edge_index[0], pad_src]).reshape(NW, EDGE_CHUNKS, 1, CH)
    dst = jnp.concatenate(
        [edge_index[1], pad_dst]).reshape(NW, EDGE_CHUNKS, 1, CH)
    ei = jnp.concatenate([src, dst], axis=2)
    zeros = jnp.zeros((ZROWS, HIDDEN), jnp.float32)

    part, h_pad = _edge_scatter_emb(emb, x_pad, ei, zeros)
    h = _gru_tc(part, h_pad, W_ih[0].T, W_hh[0].T,
                b_ih[0][None, :], b_hh[0][None, :])
    part = _edge_scatter(h, ei, zeros)
    out2 = _gru_tail_tc(part, h, W_ih[1].T, W_hh[1].T,
                        b_ih[1][None, :], b_hh[1][None, :],
                        batch[:, None], dense_W.T, dense_b[None, :],
                        clf_W.T, clf_b[None, :])
    return out2[:, 0]


# speedup vs baseline: 1.0072x; 1.0072x over previous
import functools

import jax
import jax.numpy as jnp
import numpy as np
from jax import lax
from jax.experimental import pallas as pl
from jax.experimental.pallas import tpu as pltpu
from jax.experimental.pallas import tpu_sc as plsc

N_NODES = 10000
N_EDGES = 320000
HIDDEN = 128
GRAPHS = 32
LAYERS = 2

NC = 2
NS = 16
NW = NC * NS

CH = 128

EMB_CHUNKS = 3
EMB_PER_W = EMB_CHUNKS * CH
N_PAD = NW * EMB_PER_W

EDGE_CHUNKS = 80
EDGE_PER_W = EDGE_CHUNKS * CH
E_PAD = NW * EDGE_PER_W
NBUF = 2

ACC_ROWS = 10112
ZROWS = ACC_ROWS // NS
CPROWS = 624
CPREM = N_NODES - NS * CPROWS

_sc_mesh = plsc.VectorSubcoreMesh(core_axis_name="c", subcore_axis_name="s")


NI = 4
NIF = 8
XAH = 5


@functools.partial(
    pl.kernel,
    out_type=[jax.ShapeDtypeStruct((NC, N_NODES, HIDDEN), jnp.float32),
              jax.ShapeDtypeStruct((N_PAD, HIDDEN), jnp.float32)],
    mesh=_sc_mesh,
    scratch_types=(
        [pltpu.VMEM_SHARED((ACC_ROWS, HIDDEN), jnp.float32)]
        + [pltpu.VMEM((2, CH), jnp.int32)] * NIF
        + [pltpu.VMEM((CH,), jnp.int32)] * NIF
        + [pltpu.VMEM((CH, HIDDEN), jnp.float32)] * NBUF
        + [pltpu.SemaphoreType.DMA] * NIF
        + [pltpu.SemaphoreType.DMA] * NIF
        + [pltpu.SemaphoreType.DMA] * NBUF
        + [pltpu.SemaphoreType.DMA, pltpu.SemaphoreType.DMA]
    ),
)
def _edge_scatter_emb(emb_hbm, x_hbm, ei_hbm, zeros_hbm, out_hbm, h_hbm,
                      accum, *bufs):
    idxb = bufs[:NIF]
    xs = bufs[NIF:2 * NIF]
    rows = bufs[2 * NIF:2 * NIF + NBUF]
    isem = bufs[2 * NIF + NBUF:3 * NIF + NBUF]
    xsem = bufs[3 * NIF + NBUF:4 * NIF + NBUF]
    gsem = bufs[4 * NIF + NBUF:4 * NIF + 2 * NBUF]
    ssem = bufs[4 * NIF + 2 * NBUF]
    zsem = bufs[4 * NIF + 2 * NBUF + 1]
    c = lax.axis_index("c")
    s = lax.axis_index("s")
    wid = s * NC + c

    zslice = accum.at[pl.ds(pl.multiple_of(s * ZROWS, 8), ZROWS)]
    zdesc = pltpu.async_copy(zeros_hbm, zslice, zsem)

    for t in range(NIF):
        pltpu.async_copy(ei_hbm.at[wid, t], idxb[t], isem[t])
    for t in range(XAH):
        pltpu.make_async_copy(ei_hbm.at[wid, t], idxb[t], isem[t]).wait()
        pltpu.async_copy(x_hbm.at[idxb[t].at[0]], xs[t], xsem[t])
    for b in range(NBUF):
        pltpu.make_async_copy(x_hbm.at[idxb[b].at[0]], xs[b],
                              xsem[b]).wait()
        pltpu.async_copy(emb_hbm.at[xs[b]], rows[b], gsem[b])

    zdesc.wait()
    plsc.subcore_barrier()

    def step(o, carry):
        for t in range(NIF):
            j = o * NIF + t
            b = t % NBUF
            pltpu.make_async_copy(emb_hbm.at[xs[t]], rows[b],
                                  gsem[b]).wait()
            pltpu.sync_copy(rows[b], accum.at[idxb[t].at[1]], add=True)

            @pl.when(j + NIF < EDGE_CHUNKS)
            def _reidx():
                pltpu.async_copy(ei_hbm.at[wid, j + NIF], idxb[t], isem[t])

            @pl.when(j + XAH < EDGE_CHUNKS)
            def _rexs():
                tn = (t + XAH) % NIF
                pltpu.make_async_copy(ei_hbm.at[wid, j + XAH],
                                      idxb[tn], isem[tn]).wait()
                pltpu.async_copy(x_hbm.at[idxb[tn].at[0]], xs[tn],
                                 xsem[tn])

            @pl.when(j + NBUF < EDGE_CHUNKS)
            def _regather():
                tg = (t + NBUF) % NIF
                pltpu.make_async_copy(x_hbm.at[idxb[tg].at[0]], xs[tg],
                                      xsem[tg]).wait()
                pltpu.async_copy(emb_hbm.at[xs[tg]], rows[b], gsem[b])
        return carry

    lax.fori_loop(0, EDGE_CHUNKS // NIF, step, 0)

    hbase = pl.multiple_of(wid * EMB_PER_W, CH)
    for k in range(EMB_CHUNKS):
        pltpu.async_copy(x_hbm.at[pl.ds(hbase + k * CH, CH)], xs[k],
                         isem[k])
    for k in range(NBUF):
        pltpu.make_async_copy(x_hbm.at[pl.ds(hbase + k * CH, CH)], xs[k],
                              isem[k]).wait()
        pltpu.async_copy(emb_hbm.at[xs[k]], rows[k], gsem[k])

    plsc.subcore_barrier()

    r0 = pl.multiple_of(s * CPROWS, 8)
    pltpu.sync_copy(accum.at[pl.ds(r0, CPROWS)],
                    out_hbm.at[c, pl.ds(r0, CPROWS)])

    @pl.when(s == 0)
    def _rem0():
        pltpu.sync_copy(accum.at[pl.ds(NS * CPROWS, CPREM)],
                        out_hbm.at[c, pl.ds(NS * CPROWS, CPREM)])

    for k in range(EMB_CHUNKS):
        b = k % NBUF
        o = pl.multiple_of(hbase + k * CH, CH)
        pltpu.make_async_copy(emb_hbm.at[xs[k]], rows[b], gsem[b]).wait()
        pltpu.sync_copy(rows[b], h_hbm.at[pl.ds(o, CH)])
        kn = k + NBUF
        if kn < EMB_CHUNKS:
            pltpu.make_async_copy(
                x_hbm.at[pl.ds(hbase + kn * CH, CH)], xs[kn],
                isem[kn]).wait()
            pltpu.async_copy(emb_hbm.at[xs[kn]], rows[b], gsem[b])


@functools.partial(
    pl.kernel,
    out_type=jax.ShapeDtypeStruct((NC, N_NODES, HIDDEN), jnp.float32),
    mesh=_sc_mesh,
    scratch_types=(
        [pltpu.VMEM_SHARED((ACC_ROWS, HIDDEN), jnp.float32)]
        + [pltpu.VMEM((2, CH), jnp.int32)] * NI
        + [pltpu.VMEM((CH, HIDDEN), jnp.float32)] * NBUF
        + [pltpu.SemaphoreType.DMA] * NI
        + [pltpu.SemaphoreType.DMA] * NBUF
        + [pltpu.SemaphoreType.DMA, pltpu.SemaphoreType.DMA]
    ),
)
def _edge_scatter(h_hbm, ei_hbm, zeros_hbm, out_hbm, accum, *bufs):
    idxb = bufs[:NI]
    rows = bufs[NI:NI + NBUF]
    isem = bufs[NI + NBUF:2 * NI + NBUF]
    gsem = bufs[2 * NI + NBUF:2 * NI + 2 * NBUF]
    ssem = bufs[2 * NI + 2 * NBUF]
    zsem = bufs[2 * NI + 2 * NBUF + 1]
    c = lax.axis_index("c")
    s = lax.axis_index("s")
    wid = s * NC + c

    zslice = accum.at[pl.ds(pl.multiple_of(s * ZROWS, 8), ZROWS)]
    zdesc = pltpu.async_copy(zeros_hbm, zslice, zsem)

    for t in range(NI):
        pltpu.async_copy(ei_hbm.at[wid, t], idxb[t], isem[t])
    for b in range(NBUF):
        pltpu.make_async_copy(ei_hbm.at[wid, b], idxb[b], isem[b]).wait()
        pltpu.async_copy(h_hbm.at[idxb[b].at[0]], rows[b], gsem[b])

    zdesc.wait()
    plsc.subcore_barrier()

    def step(o, carry):
        for t in range(NI):
            j = o * NI + t
            b = t % NBUF
            pltpu.make_async_copy(h_hbm.at[idxb[t].at[0]], rows[b],
                                  gsem[b]).wait()
            pltpu.sync_copy(rows[b], accum.at[idxb[t].at[1]], add=True)

            @pl.when(j + NI < EDGE_CHUNKS)
            def _reidx():
                pltpu.async_copy(ei_hbm.at[wid, j + NI], idxb[t], isem[t])

            @pl.when(j + NBUF < EDGE_CHUNKS)
            def _regather():
                tn = (t + NBUF) % NI
                pltpu.make_async_copy(ei_hbm.at[wid, j + NBUF], idxb[tn],
                                      isem[tn]).wait()
                pltpu.async_copy(h_hbm.at[idxb[tn].at[0]], rows[b],
                                 gsem[b])
        return carry

    lax.fori_loop(0, EDGE_CHUNKS // NI, step, 0)
    plsc.subcore_barrier()

    r0 = pl.multiple_of(s * CPROWS, 8)
    pltpu.sync_copy(accum.at[pl.ds(r0, CPROWS)],
                    out_hbm.at[c, pl.ds(r0, CPROWS)])

    @pl.when(s == 0)
    def _rem():
        pltpu.sync_copy(accum.at[pl.ds(NS * CPROWS, CPREM)],
                        out_hbm.at[c, pl.ds(NS * CPROWS, CPREM)])


_GRID_R = 1000


def _gru_block(p0, p1, h, wih, whh, bih, bhh):
    xn = p0 + p1
    gi = jnp.dot(xn, wih, preferred_element_type=jnp.float32) + bih
    gh = jnp.dot(h, whh, preferred_element_type=jnp.float32) + bhh
    r = jax.nn.sigmoid(gi[:, :HIDDEN] + gh[:, :HIDDEN])
    z = jax.nn.sigmoid(gi[:, HIDDEN:2 * HIDDEN] + gh[:, HIDDEN:2 * HIDDEN])
    n = jnp.tanh(gi[:, 2 * HIDDEN:] + r * gh[:, 2 * HIDDEN:])
    return (1.0 - z) * n + z * h


def _gru_body(p0_ref, p1_ref, h_ref, wih_ref, whh_ref, bih_ref, bhh_ref,
              out_ref):
    out_ref[...] = _gru_block(p0_ref[0], p1_ref[0], h_ref[...],
                              wih_ref[...], whh_ref[...],
                              bih_ref[...], bhh_ref[...])


_P_SPEC0 = pl.BlockSpec((1, 1000, HIDDEN), lambda i: (0, i, 0))
_P_SPEC1 = pl.BlockSpec((1, 1000, HIDDEN), lambda i: (1, i, 0))


def _gru_tc(part, h, wih_t, whh_t, bih, bhh):
    grid = (N_NODES // _GRID_R,)
    blk = lambda i: (i, 0)
    whole = lambda i: (0, 0)
    return pl.pallas_call(
        _gru_body,
        grid=grid,
        in_specs=[
            _P_SPEC0,
            _P_SPEC1,
            pl.BlockSpec((_GRID_R, HIDDEN), blk),
            pl.BlockSpec((HIDDEN, 3 * HIDDEN), whole),
            pl.BlockSpec((HIDDEN, 3 * HIDDEN), whole),
            pl.BlockSpec((1, 3 * HIDDEN), whole),
            pl.BlockSpec((1, 3 * HIDDEN), whole),
        ],
        out_specs=pl.BlockSpec((_GRID_R, HIDDEN), blk),
        out_shape=jax.ShapeDtypeStruct((N_NODES, HIDDEN), jnp.float32),
    )(part, part, h, wih_t, whh_t, bih, bhh)


def _gru_tail_body(p0_ref, p1_ref, h_ref, wih_ref, whh_ref, bih_ref,
                   bhh_ref, bat_ref, dw_ref, db_ref, cw_ref, cb_ref,
                   out_ref, pooled_ref):
    i = pl.program_id(0)

    @pl.when(i == 0)
    def _init():
        pooled_ref[...] = jnp.full((GRAPHS, HIDDEN), -jnp.inf,
                                   dtype=jnp.float32)

    hn = _gru_block(p0_ref[0], p1_ref[0], h_ref[...],
                    wih_ref[...], whh_ref[...],
                    bih_ref[...], bhh_ref[...])
    hd = jnp.dot(hn, dw_ref[...], preferred_element_type=jnp.float32)
    hd = hd + db_ref[...]
    bat = bat_ref[...]
    neg = jnp.float32(-jnp.inf)
    zero = jnp.float32(0.0)
    for g in range(GRAPHS):
        madd = jnp.where(bat == g, zero, neg)
        m = (hd + madd).max(axis=0, keepdims=True)
        pooled_ref[g:g + 1, :] = jnp.maximum(pooled_ref[g:g + 1, :], m)

    @pl.when(i == pl.num_programs(0) - 1)
    def _fin():
        logits = jnp.dot(pooled_ref[...], cw_ref[...],
                         preferred_element_type=jnp.float32) + cb_ref[...]
        out_ref[...] = jax.nn.sigmoid(logits)


def _gru_tail_tc(part, h, wih_t, whh_t, bih, bhh, bat2d, dw_t, db, cw_t, cb):
    grid = (N_NODES // _GRID_R,)
    blk = lambda i: (i, 0)
    whole = lambda i: (0, 0)
    return pl.pallas_call(
        _gru_tail_body,
        grid=grid,
        in_specs=[
            _P_SPEC0,
            _P_SPEC1,
            pl.BlockSpec((_GRID_R, HIDDEN), blk),
            pl.BlockSpec((HIDDEN, 3 * HIDDEN), whole),
            pl.BlockSpec((HIDDEN, 3 * HIDDEN), whole),
            pl.BlockSpec((1, 3 * HIDDEN), whole),
            pl.BlockSpec((1, 3 * HIDDEN), whole),
            pl.BlockSpec((_GRID_R, 1), blk),
            pl.BlockSpec((HIDDEN, HIDDEN), whole),
            pl.BlockSpec((1, HIDDEN), whole),
            pl.BlockSpec((HIDDEN, 1), whole),
            pl.BlockSpec((1, 1), whole),
        ],
        out_specs=pl.BlockSpec((GRAPHS, 1), whole),
        out_shape=jax.ShapeDtypeStruct((GRAPHS, 1), jnp.float32),
        scratch_shapes=[pltpu.VMEM((GRAPHS, HIDDEN), jnp.float32)],
    )(part, part, h, wih_t, whh_t, bih, bhh, bat2d, dw_t, db, cw_t, cb)


def kernel(x, edge_index, batch, emb, W_ih, W_hh, b_ih, b_hh,
           dense_W, dense_b, clf_W, clf_b):
    x_pad = jnp.concatenate(
        [x, jnp.zeros((N_PAD - N_NODES,), jnp.int32)])

    npad = E_PAD - N_EDGES
    pad_iota = np.arange(npad, dtype=np.int32)
    pad_src = jnp.asarray(pad_iota % N_NODES)
    pad_dst = jnp.asarray(N_NODES + pad_iota % (ACC_ROWS - N_NODES))
    src = jnp.concatenate(
        [edge_index[0], pad_src]).reshape(NW, EDGE_CHUNKS, 1, CH)
    dst = jnp.concatenate(
        [edge_index[1], pad_dst]).reshape(NW, EDGE_CHUNKS, 1, CH)
    ei = jnp.concatenate([src, dst], axis=2)
    zeros = jnp.zeros((ZROWS, HIDDEN), jnp.float32)

    part, h_pad = _edge_scatter_emb(emb, x_pad, ei, zeros)
    h = _gru_tc(part, h_pad, W_ih[0].T, W_hh[0].T,
                b_ih[0][None, :], b_hh[0][None, :])
    part = _edge_scatter(h, ei, zeros)
    out2 = _gru_tail_tc(part, h, W_ih[1].T, W_hh[1].T,
                        b_ih[1][None, :], b_hh[1][None, :],
                        batch[:, None], dense_W.T, dense_b[None, :],
                        clf_W.T, clf_b[None, :])
    return out2[:, 0]
